# speedup vs baseline: 1.4992x; 1.4992x over previous
import jax
import jax.numpy as jnp
from jax import lax
from jax.experimental import pallas as pl
from jax.experimental.pallas import tpu as pltpu
from jax.experimental.pallas import tpu_sc as plsc

N_ATOMS = 100000
EMBED = 128
CHUNK = 128
NUM_FULL = N_ATOMS // CHUNK
TAIL = N_ATOMS - NUM_FULL * CHUNK
TAIL_BASE = NUM_FULL * CHUNK
NC, NS = 2, 16
NW = NC * NS
STEPS = (NUM_FULL + NW - 1) // NW


def _body(table_hbm, idx_hbm, out_hbm, idx_v, rows_v, idx_t, rows_t, sem):
    w = lax.axis_index("s") * NC + lax.axis_index("c")

    def step(j, carry):
        c = j * NW + w

        @pl.when(c < NUM_FULL)
        def _():
            base = pl.multiple_of(c * CHUNK, CHUNK)
            pltpu.sync_copy(idx_hbm.at[pl.ds(base, CHUNK)], idx_v)
            pltpu.async_copy(table_hbm.at[idx_v], rows_v, sem).wait()
            pltpu.sync_copy(rows_v, out_hbm.at[pl.ds(base, CHUNK)])

        return carry

    lax.fori_loop(0, STEPS, step, None)

    @pl.when(w == NW - 1)
    def _():
        pltpu.sync_copy(idx_hbm.at[pl.ds(TAIL_BASE, TAIL)], idx_t)
        pltpu.async_copy(table_hbm.at[idx_t], rows_t, sem).wait()
        pltpu.sync_copy(rows_t, out_hbm.at[pl.ds(TAIL_BASE, TAIL)])


def kernel(atomic_numbers, embedding_table):
    k = pl.kernel(
        _body,
        out_type=jax.ShapeDtypeStruct((N_ATOMS, EMBED), jnp.float32),
        mesh=plsc.VectorSubcoreMesh(
            core_axis_name="c", subcore_axis_name="s",
            num_cores=NC, num_subcores=NS,
        ),
        scratch_types=[
            pltpu.VMEM((CHUNK,), jnp.int32),
            pltpu.VMEM((CHUNK, EMBED), jnp.float32),
            pltpu.VMEM((TAIL,), jnp.int32),
            pltpu.VMEM((TAIL, EMBED), jnp.float32),
            pltpu.SemaphoreType.DMA,
        ],
    )
    return k(embedding_table, atomic_numbers.astype(jnp.int32))


# speedup vs baseline: 1.5009x; 1.0011x over previous
import jax
import jax.numpy as jnp
from jax import lax
from jax.experimental import pallas as pl
from jax.experimental.pallas import tpu as pltpu
from jax.experimental.pallas import tpu_sc as plsc

N_ATOMS = 100000
EMBED = 128
CHUNK = 128
NUM_FULL = N_ATOMS // CHUNK
TAIL = N_ATOMS - NUM_FULL * CHUNK
TAIL_BASE = NUM_FULL * CHUNK
NC, NS = 2, 16
NW = NC * NS
PER_W = 26
NCH_PAD = PER_W * NW
PAIRS = PER_W // 2
TAIL_SLOT = (NUM_FULL - 13) // NW
TAIL_W = NUM_FULL - TAIL_SLOT * NW


def _body(table_hbm, idx_hbm, out_hbm, idx_l, rows_a, rows_b,
          gsem_a, gsem_b, ssem_a, ssem_b):
    w = lax.axis_index("s") * NC + lax.axis_index("c")

    pltpu.sync_copy(idx_hbm.at[w], idx_l)

    def pair(j, carry):
        def do(k, rows, gsem, ssem):
            c = k * NW + w

            @pl.when(c < NUM_FULL)
            def _():
                @pl.when(j > 0)
                def _():
                    pltpu.make_async_copy(
                        rows, out_hbm.at[pl.ds(0, CHUNK)], ssem).wait()
                pltpu.async_copy(table_hbm.at[idx_l.at[k]], rows, gsem).wait()
                base = pl.multiple_of(c * CHUNK, CHUNK)
                pltpu.async_copy(rows, out_hbm.at[pl.ds(base, CHUNK)], ssem)

        do(2 * j, rows_a, gsem_a, ssem_a)
        do(2 * j + 1, rows_b, gsem_b, ssem_b)
        return carry

    lax.fori_loop(0, PAIRS, pair, None)

    pltpu.make_async_copy(rows_a, out_hbm.at[pl.ds(0, CHUNK)], ssem_a).wait()
    pltpu.make_async_copy(rows_b, out_hbm.at[pl.ds(0, CHUNK)], ssem_b).wait()

    @pl.when(w == TAIL_W)
    def _():
        pltpu.async_copy(table_hbm.at[idx_l.at[TAIL_SLOT]], rows_a,
                         gsem_a).wait()
        pltpu.sync_copy(rows_a.at[pl.ds(0, TAIL)],
                        out_hbm.at[pl.ds(TAIL_BASE, TAIL)])


def kernel(atomic_numbers, embedding_table):
    idx = atomic_numbers.astype(jnp.int32)
    pad = jnp.zeros((NCH_PAD * CHUNK - N_ATOMS,), jnp.int32)
    idx3 = jnp.concatenate([idx, pad]).reshape(PER_W, NW, CHUNK).swapaxes(0, 1)

    k = pl.kernel(
        _body,
        out_type=jax.ShapeDtypeStruct((N_ATOMS, EMBED), jnp.float32),
        mesh=plsc.VectorSubcoreMesh(
            core_axis_name="c", subcore_axis_name="s",
            num_cores=NC, num_subcores=NS,
        ),
        scratch_types=[
            pltpu.VMEM((PER_W, CHUNK), jnp.int32),
            pltpu.VMEM((CHUNK, EMBED), jnp.float32),
            pltpu.VMEM((CHUNK, EMBED), jnp.float32),
            pltpu.SemaphoreType.DMA,
            pltpu.SemaphoreType.DMA,
            pltpu.SemaphoreType.DMA,
            pltpu.SemaphoreType.DMA,
        ],
    )
    return k(embedding_table, idx3)


# speedup vs baseline: 1.5024x; 1.0010x over previous
import jax
import jax.numpy as jnp
from jax import lax
from jax.experimental import pallas as pl
from jax.experimental.pallas import tpu as pltpu
from jax.experimental.pallas import tpu_sc as plsc

N_ATOMS = 100000
EMBED = 128
CHUNK = 128
NUM_FULL = N_ATOMS // CHUNK
TAIL = N_ATOMS - NUM_FULL * CHUNK
TAIL_BASE = NUM_FULL * CHUNK
NC, NS = 2, 16
NW = NC * NS
PER_W = 26
NCH_PAD = PER_W * NW
PAIRS = PER_W // 2
TAIL_SLOT = (NUM_FULL - 13) // NW
TAIL_W = NUM_FULL - TAIL_SLOT * NW


def _body(table_hbm, idx_hbm, out_hbm, idx_l, rows_a, rows_b,
          gsem_a, gsem_b, ssem_a, ssem_b):
    w = lax.axis_index("s") * NC + lax.axis_index("c")

    pltpu.sync_copy(idx_hbm.at[w], idx_l)

    def pair(j, carry):
        def start(k, rows, gsem, ssem):
            c = k * NW + w

            @pl.when(c < NUM_FULL)
            def _():
                @pl.when(j > 0)
                def _():
                    pltpu.make_async_copy(
                        rows, out_hbm.at[pl.ds(0, CHUNK)], ssem).wait()
                pltpu.async_copy(table_hbm.at[idx_l.at[k]], rows, gsem)

        def finish(k, rows, gsem, ssem):
            c = k * NW + w

            @pl.when(c < NUM_FULL)
            def _():
                pltpu.make_async_copy(
                    table_hbm.at[idx_l.at[k]], rows, gsem).wait()
                base = pl.multiple_of(c * CHUNK, CHUNK)
                pltpu.async_copy(rows, out_hbm.at[pl.ds(base, CHUNK)], ssem)

        start(2 * j, rows_a, gsem_a, ssem_a)
        start(2 * j + 1, rows_b, gsem_b, ssem_b)
        finish(2 * j, rows_a, gsem_a, ssem_a)
        finish(2 * j + 1, rows_b, gsem_b, ssem_b)
        return carry

    lax.fori_loop(0, PAIRS, pair, None)

    pltpu.make_async_copy(rows_a, out_hbm.at[pl.ds(0, CHUNK)], ssem_a).wait()
    pltpu.make_async_copy(rows_b, out_hbm.at[pl.ds(0, CHUNK)], ssem_b).wait()

    @pl.when(w == TAIL_W)
    def _():
        pltpu.async_copy(table_hbm.at[idx_l.at[TAIL_SLOT]], rows_a,
                         gsem_a).wait()
        pltpu.sync_copy(rows_a.at[pl.ds(0, TAIL)],
                        out_hbm.at[pl.ds(TAIL_BASE, TAIL)])


def kernel(atomic_numbers, embedding_table):
    idx = atomic_numbers.astype(jnp.int32)
    pad = jnp.zeros((NCH_PAD * CHUNK - N_ATOMS,), jnp.int32)
    idx3 = jnp.concatenate([idx, pad]).reshape(PER_W, NW, CHUNK).swapaxes(0, 1)

    k = pl.kernel(
        _body,
        out_type=jax.ShapeDtypeStruct((N_ATOMS, EMBED), jnp.float32),
        mesh=plsc.VectorSubcoreMesh(
            core_axis_name="c", subcore_axis_name="s",
            num_cores=NC, num_subcores=NS,
        ),
        scratch_types=[
            pltpu.VMEM((PER_W, CHUNK), jnp.int32),
            pltpu.VMEM((CHUNK, EMBED), jnp.float32),
            pltpu.VMEM((CHUNK, EMBED), jnp.float32),
            pltpu.SemaphoreType.DMA,
            pltpu.SemaphoreType.DMA,
            pltpu.SemaphoreType.DMA,
            pltpu.SemaphoreType.DMA,
        ],
    )
    return k(embedding_table, idx3)


# speedup vs baseline: 4.5200x; 3.0084x over previous
import jax
import jax.numpy as jnp
from jax import lax
from jax.experimental import pallas as pl
from jax.experimental.pallas import tpu as pltpu
from jax.experimental.pallas import tpu_sc as plsc

N_ATOMS = 100000
EMBED = 128
CHUNK = 128
NUM_FULL = N_ATOMS // CHUNK
TAIL = N_ATOMS - NUM_FULL * CHUNK
TAIL_BASE = NUM_FULL * CHUNK
NC, NS = 2, 16
NW = NC * NS
PER_W = 26
NCH_PAD = PER_W * NW
PAIRS = PER_W // 2
TAIL_SLOT = (NUM_FULL - 13) // NW
TAIL_W = NUM_FULL - TAIL_SLOT * NW


def _body(table_hbm, idx_hbm, out_hbm, table_v, idx_l, rows_a, rows_b,
          gsem_a, gsem_b, ssem_a, ssem_b):
    w = lax.axis_index("s") * NC + lax.axis_index("c")

    @pl.when(lax.axis_index("s") == 0)
    def _():
        pltpu.sync_copy(table_hbm, table_v)
    pltpu.sync_copy(idx_hbm.at[w], idx_l)
    plsc.subcore_barrier()

    def pair(j, carry):
        def start(k, rows, gsem, ssem):
            c = k * NW + w

            @pl.when(c < NUM_FULL)
            def _():
                @pl.when(j > 0)
                def _():
                    pltpu.make_async_copy(
                        rows, out_hbm.at[pl.ds(0, CHUNK)], ssem).wait()
                pltpu.async_copy(table_v.at[idx_l.at[k]], rows, gsem)

        def finish(k, rows, gsem, ssem):
            c = k * NW + w

            @pl.when(c < NUM_FULL)
            def _():
                pltpu.make_async_copy(
                    table_v.at[idx_l.at[k]], rows, gsem).wait()
                base = pl.multiple_of(c * CHUNK, CHUNK)
                pltpu.async_copy(rows, out_hbm.at[pl.ds(base, CHUNK)], ssem)

        start(2 * j, rows_a, gsem_a, ssem_a)
        start(2 * j + 1, rows_b, gsem_b, ssem_b)
        finish(2 * j, rows_a, gsem_a, ssem_a)
        finish(2 * j + 1, rows_b, gsem_b, ssem_b)
        return carry

    lax.fori_loop(0, PAIRS, pair, None)

    pltpu.make_async_copy(rows_a, out_hbm.at[pl.ds(0, CHUNK)], ssem_a).wait()
    pltpu.make_async_copy(rows_b, out_hbm.at[pl.ds(0, CHUNK)], ssem_b).wait()

    @pl.when(w == TAIL_W)
    def _():
        pltpu.async_copy(table_v.at[idx_l.at[TAIL_SLOT]], rows_a,
                         gsem_a).wait()
        pltpu.sync_copy(rows_a.at[pl.ds(0, TAIL)],
                        out_hbm.at[pl.ds(TAIL_BASE, TAIL)])


def kernel(atomic_numbers, embedding_table):
    idx = atomic_numbers.astype(jnp.int32)
    pad = jnp.zeros((NCH_PAD * CHUNK - N_ATOMS,), jnp.int32)
    idx3 = jnp.concatenate([idx, pad]).reshape(PER_W, NW, CHUNK).swapaxes(0, 1)

    k = pl.kernel(
        _body,
        out_type=jax.ShapeDtypeStruct((N_ATOMS, EMBED), jnp.float32),
        mesh=plsc.VectorSubcoreMesh(
            core_axis_name="c", subcore_axis_name="s",
            num_cores=NC, num_subcores=NS,
        ),
        scratch_types=[
            pltpu.VMEM_SHARED((120, EMBED), jnp.float32),
            pltpu.VMEM((PER_W, CHUNK), jnp.int32),
            pltpu.VMEM((CHUNK, EMBED), jnp.float32),
            pltpu.VMEM((CHUNK, EMBED), jnp.float32),
            pltpu.SemaphoreType.DMA,
            pltpu.SemaphoreType.DMA,
            pltpu.SemaphoreType.DMA,
            pltpu.SemaphoreType.DMA,
        ],
    )
    return k(embedding_table, idx3)
